# Initial kernel scaffold; baseline (speedup 1.0000x reference)
import jax
import jax.numpy as jnp
from jax.experimental import pallas as pl


def kernel(embedding, study_indexes, W, b):
    raise NotImplementedError("write your pallas kernel here")


# speedup vs baseline: 2.4954x; 2.4954x over previous
import functools

import jax
import jax.numpy as jnp
from jax import lax
from jax.experimental import pallas as pl
from jax.experimental.pallas import tpu as pltpu
from jax.experimental.pallas import tpu_sc as plsc

N = 160000
D = 256
S = 10000
NC = 2
NS = 16
NW = NC * NS
RPW = N // NW
C = 200
KCH = RPW // C
NK = D // 16
RING = 4
PW = D + 32


def _main_body(emb_hbm, idx_hbm, w_hbm, b_hbm, out_hbm, part_hbm,
               ebuf, ibuf, wbuf, bbuf, pbuf, fbuf, accbuf, pstage, sem):
    c = lax.axis_index("c")
    sub = lax.axis_index("s")
    wid = sub * NC + c
    row0 = wid * RPW

    pltpu.sync_copy(w_hbm, wbuf)
    pltpu.sync_copy(b_hbm, bbuf)

    @pl.when(wid > 0)
    def _():
        off = pl.multiple_of(row0 - 16, 8)
        pltpu.sync_copy(idx_hbm.at[pl.ds(off, 16)], pbuf)

    bv = bbuf[...]
    wv = [wbuf[pl.ds(16 * k, 16)] for k in range(NK)]
    zero16 = jnp.zeros((16,), jnp.float32)

    for k in range(NK + 1):
        accbuf[pl.ds(16 * k, 16)] = zero16

    prev0 = jnp.where(wid > 0, pbuf[...][15], jnp.int32(-1))

    def drain_one():
        pltpu.make_async_copy(out_hbm.at[0], fbuf.at[0], sem).wait()

    def write_partial(slot, seg):
        for k in range(NK + 1):
            pstage[pl.ds(16 * k, 16)] = accbuf[pl.ds(16 * k, 16)]
        pstage[pl.ds(16 * (NK + 1), 16)] = jnp.full(
            (16,), seg, jnp.int32).astype(jnp.float32)
        pltpu.sync_copy(pstage, part_hbm.at[slot])

    def reset_acc():
        for k in range(NK + 1):
            accbuf[pl.ds(16 * k, 16)] = zero16

    def flush_interior(fcount, seg):
        slot = fcount & (RING - 1)

        @pl.when((slot == 0) & (fcount > 0))
        def _():
            for _ in range(RING):
                drain_one()

        inv = 1.0 / accbuf[pl.ds(16 * NK, 16)]
        for k in range(NK):
            fbuf[slot, pl.ds(16 * k, 16)] = accbuf[pl.ds(16 * k, 16)] * inv
        pltpu.async_copy(fbuf.at[slot], out_hbm.at[seg], sem)
        reset_acc()

    def chunk_body(kc, st):
        prev, nb, fcount = st
        p = pl.multiple_of(row0 + kc * C, 8)
        pltpu.sync_copy(emb_hbm.at[pl.ds(p, C)], ebuf)
        pltpu.sync_copy(idx_hbm.at[pl.ds(p, C)], ibuf.at[pl.ds(0, C)])

        def row_body(r, rst):
            prev, nb, fcount = rst
            sid = ibuf[pl.ds(r, 16)][0]
            boundary = sid != prev

            @pl.when(boundary & (nb == 0))
            def _():
                write_partial(2 * wid, prev)
                reset_acc()

            @pl.when(boundary & (nb > 0))
            def _():
                flush_interior(fcount, prev)

            nb_n = nb + jnp.where(boundary, 1, 0).astype(jnp.int32)
            fcount_n = fcount + jnp.where(boundary & (nb > 0), 1, 0).astype(
                jnp.int32)

            e = [ebuf[r, pl.ds(16 * k, 16)] for k in range(NK)]
            d0 = e[0] * wv[0]
            d1 = e[1] * wv[1]
            d2 = e[2] * wv[2]
            d3 = e[3] * wv[3]
            for k in range(4, NK, 4):
                d0 = d0 + e[k] * wv[k]
                d1 = d1 + e[k + 1] * wv[k + 1]
                d2 = d2 + e[k + 2] * wv[k + 2]
                d3 = d3 + e[k + 3] * wv[k + 3]
            dvec = (d0 + d1) + (d2 + d3)
            dot = jnp.sum(dvec)
            attn = jnp.exp(jnp.full((16,), dot, jnp.float32) + bv)
            for k in range(NK):
                plsc.addupdate(accbuf.at[pl.ds(16 * k, 16)], attn * e[k])
            plsc.addupdate(accbuf.at[pl.ds(16 * NK, 16)], attn)
            return (sid, nb_n, fcount_n)

        return lax.fori_loop(0, C, row_body, (prev, nb, fcount))

    prev, nb, fcount = lax.fori_loop(
        0, KCH, chunk_body, (prev0, jnp.int32(0), jnp.int32(0)))

    @pl.when(nb == 0)
    def _():
        write_partial(2 * wid, prev)
        reset_acc()
        write_partial(2 * wid + 1, jnp.int32(-1))

    @pl.when(nb > 0)
    def _():
        write_partial(2 * wid + 1, prev)

    nout = jnp.where(fcount == 0, jnp.int32(0),
                     ((fcount - 1) & (RING - 1)) + 1)
    for i in range(RING):
        @pl.when(i < nout)
        def _():
            drain_one()


def _merge_body(part_hbm, out_hbm, mp, macc, fstage, sem):
    c = lax.axis_index("c")
    sub = lax.axis_index("s")
    wid = sub * NC + c

    @pl.when(wid == 0)
    def _():
        pltpu.sync_copy(part_hbm, mp)
        zero16 = jnp.zeros((16,), jnp.float32)
        for k in range(NK + 1):
            macc[pl.ds(16 * k, 16)] = zero16

        def finalize(seg):
            inv = 1.0 / macc[pl.ds(16 * NK, 16)]
            for k in range(NK):
                fstage[pl.ds(16 * k, 16)] = macc[pl.ds(16 * k, 16)] * inv
            pltpu.sync_copy(fstage, out_hbm.at[seg])
            for k in range(NK + 1):
                macc[pl.ds(16 * k, 16)] = zero16

        def rec_body(i, cur):
            segf = mp[i, pl.ds(16 * (NK + 1), 16)]
            seg = segf.astype(jnp.int32)[0]
            valid = seg >= 0
            newseg = valid & (seg != cur)

            @pl.when(newseg & (cur >= 0))
            def _():
                finalize(cur)

            @pl.when(valid)
            def _():
                for k in range(NK + 1):
                    sl = pl.ds(16 * k, 16)
                    macc[sl] = macc[sl] + mp[i, sl]

            return jnp.where(valid, seg, cur)

        cur = lax.fori_loop(0, 2 * NW, rec_body, jnp.int32(-1))

        @pl.when(cur >= 0)
        def _():
            finalize(cur)


@jax.jit
def _run(embedding, idx32, w_flat, b_vec):
    mesh = plsc.VectorSubcoreMesh(core_axis_name="c", subcore_axis_name="s")
    params = pltpu.CompilerParams(needs_layout_passes=False)
    main_k = pl.kernel(
        _main_body,
        out_type=jax.ShapeDtypeStruct((2 * NW, PW), jnp.float32),
        mesh=mesh,
        compiler_params=params,
        scratch_types=[
            pltpu.VMEM((C, D), jnp.float32),
            pltpu.VMEM((C + 16,), jnp.int32),
            pltpu.VMEM((D,), jnp.float32),
            pltpu.VMEM((16,), jnp.float32),
            pltpu.VMEM((16,), jnp.int32),
            pltpu.VMEM((RING, D), jnp.float32),
            pltpu.VMEM((D + 16,), jnp.float32),
            pltpu.VMEM((PW,), jnp.float32),
            pltpu.SemaphoreType.DMA,
        ],
    )
    merge_k = pl.kernel(
        _merge_body,
        out_type=(),
        mesh=mesh,
        compiler_params=params,
        scratch_types=[
            pltpu.VMEM((2 * NW, PW), jnp.float32),
            pltpu.VMEM((D + 16,), jnp.float32),
            pltpu.VMEM((D,), jnp.float32),
            pltpu.SemaphoreType.DMA,
        ],
    )
    out_ref = jax.new_ref(jnp.zeros((S, D), jnp.float32))
    part = main_k(embedding, idx32, w_flat, b_vec, out_ref)
    merge_k(part, out_ref)
    return out_ref[...]


def kernel(embedding, study_indexes, W, b):
    idx32 = study_indexes.astype(jnp.int32)
    w_flat = W.reshape(D)
    b_vec = jnp.broadcast_to(b.astype(jnp.float32), (16,))
    return _run(embedding, idx32, w_flat, b_vec)


# speedup vs baseline: 3.7011x; 1.4831x over previous
import functools

import jax
import jax.numpy as jnp
from jax import lax
from jax.experimental import pallas as pl
from jax.experimental.pallas import tpu as pltpu
from jax.experimental.pallas import tpu_sc as plsc

N = 160000
D = 256
S = 10000
NC = 2
NS = 16
NW = NC * NS
RPW = N // NW
C = 200
KCH = RPW // C
NG = C // 16
NK = D // 16
RING = 4
PW = D + 32
IB = C + 24


def _main_body(emb_hbm, idxp_hbm, w_hbm, b_hbm, out_hbm, part_hbm,
               ebuf, ibuf, wbuf, bbuf, fbuf, accbuf, pstage,
               csem0, csem1, fsem):
    cc = lax.axis_index("c")
    sub = lax.axis_index("s")
    wid = sub * NC + cc
    row0 = wid * RPW

    pltpu.sync_copy(w_hbm, wbuf)
    pltpu.sync_copy(b_hbm, bbuf)

    bv = bbuf[...]
    wv = [wbuf[pl.ds(16 * k, 16)] for k in range(NK)]
    zero16 = jnp.zeros((16,), jnp.float32)
    pow2 = (jnp.int32(1) << jax.lax.broadcasted_iota(jnp.int32, (16,), 0))

    for k in range(NK + 1):
        accbuf[pl.ds(16 * k, 16)] = zero16

    def chunk_off(kc):
        return pl.multiple_of(row0 + kc * C, 8)

    def start_chunk(kc, par):
        p = chunk_off(kc)
        csemx = csem0 if par == 0 else csem1
        pltpu.async_copy(emb_hbm.at[pl.ds(p, C)],
                         ebuf.at[pl.ds(par * C, C)], csemx)
        pltpu.async_copy(idxp_hbm.at[pl.ds(p, C + 8)],
                         ibuf.at[pl.ds(par * IB, C + 8)], csemx)

    def wait_chunk(kc, par):
        p = chunk_off(kc)
        csemx = csem0 if par == 0 else csem1
        pltpu.make_async_copy(
            emb_hbm.at[pl.ds(p, C)], ebuf.at[pl.ds(par * C, C)], csemx).wait()
        pltpu.make_async_copy(
            idxp_hbm.at[pl.ds(p, C + 8)],
            ibuf.at[pl.ds(par * IB, C + 8)], csemx).wait()

    def drain_one():
        pltpu.make_async_copy(out_hbm.at[0], fbuf.at[0], fsem).wait()

    def reset_acc():
        for k in range(NK + 1):
            accbuf[pl.ds(16 * k, 16)] = zero16

    def write_partial(slot, seg):
        for k in range(NK + 1):
            pstage[pl.ds(16 * k, 16)] = accbuf[pl.ds(16 * k, 16)]
        pstage[pl.ds(16 * (NK + 1), 16)] = jnp.full(
            (16,), seg, jnp.int32).astype(jnp.float32)
        pltpu.sync_copy(pstage, part_hbm.at[slot])

    def flush_interior(fcount, seg):
        slot = fcount & (RING - 1)

        @pl.when((slot == 0) & (fcount > 0))
        def _():
            for _ in range(RING):
                drain_one()

        inv = 1.0 / accbuf[pl.ds(16 * NK, 16)]
        for k in range(NK):
            fbuf[slot, pl.ds(16 * k, 16)] = accbuf[pl.ds(16 * k, 16)] * inv
        pltpu.async_copy(fbuf.at[slot], out_hbm.at[seg], fsem)

    def accumulate_row(par, row):
        row2 = par * C + row
        e = [ebuf[row2, pl.ds(16 * k, 16)] for k in range(NK)]
        d0 = e[0] * wv[0]
        d1 = e[1] * wv[1]
        d2 = e[2] * wv[2]
        d3 = e[3] * wv[3]
        for k in range(4, NK, 4):
            d0 = d0 + e[k] * wv[k]
            d1 = d1 + e[k + 1] * wv[k + 1]
            d2 = d2 + e[k + 2] * wv[k + 2]
            d3 = d3 + e[k + 3] * wv[k + 3]
        dvec = (d0 + d1) + (d2 + d3)
        dot = jnp.sum(dvec)
        attn = jnp.exp(jnp.full((16,), dot, jnp.float32) + bv)
        for k in range(NK):
            plsc.addupdate(accbuf.at[pl.ds(16 * k, 16)], attn * e[k])
        plsc.addupdate(accbuf.at[pl.ds(16 * NK, 16)], attn)

    UNROLL = 4

    def chunk_body(kc, st):
        nb, fcount = st
        par = kc & 1

        @pl.when(par == 0)
        def _():
            wait_chunk(kc, 0)

        @pl.when(par == 1)
        def _():
            wait_chunk(kc, 1)

        @pl.when(kc < KCH - 1)
        def _():
            @pl.when(par == 0)
            def _():
                start_chunk(kc + 1, 1)

            @pl.when(par == 1)
            def _():
                start_chunk(kc + 1, 0)

        def group_body(gi, gst):
            base = 16 * gi
            av = ibuf[pl.ds(par * IB + base + 8, 16)]
            amv = ibuf[pl.ds(par * IB + base + 7, 16)]
            neq = av != amv
            bits_v = jnp.where(neq, pow2, jnp.int32(0))
            bits = jnp.sum(bits_v)
            bits = jnp.where(base + 16 <= C, bits, bits & 0xFF)

            def quad_body(rq, qst):
                nb, fcount = qst
                q0 = UNROLL * rq

                bjs, nbs, fcs = [], [], []
                nbq, fcq = nb, fcount
                for j in range(UNROLL):
                    bj = ((bits >> (q0 + j)) & 1) == 1
                    bjs.append(bj)
                    nbs.append(nbq)
                    fcs.append(fcq)
                    fcq = fcq + jnp.where(
                        bj & (nbq > 0), 1, 0).astype(jnp.int32)
                    nbq = nbq + jnp.where(bj, 1, 0).astype(jnp.int32)

                @pl.when(base + q0 < C)
                def _():
                    for j in range(UNROLL):
                        jj = q0 + j
                        row = base + jj

                        @pl.when(bjs[j])
                        def _():
                            seg = ibuf[pl.ds(par * IB + base + 7 + jj, 16)][0]

                            @pl.when(nbs[j] == 0)
                            def _():
                                write_partial(2 * wid, seg)

                            @pl.when(nbs[j] > 0)
                            def _():
                                flush_interior(fcs[j], seg)

                            reset_acc()

                        accumulate_row(par, row)

                nbq = jnp.where(base + q0 < C, nbq, nb)
                fcq = jnp.where(base + q0 < C, fcq, fcount)
                return nbq, fcq

            return lax.fori_loop(0, 16 // UNROLL, quad_body, gst)

        return lax.fori_loop(0, NG + 1, group_body, (nb, fcount))

    start_chunk(0, 0)
    nb, fcount = lax.fori_loop(0, KCH, chunk_body,
                               (jnp.int32(0), jnp.int32(0)))

    par_last = (KCH - 1) & 1
    seg_last = ibuf[pl.ds(par_last * IB + C - 8, 16)][15]

    @pl.when(nb == 0)
    def _():
        write_partial(2 * wid, seg_last)
        write_partial(2 * wid + 1, jnp.int32(-1))

    @pl.when(nb > 0)
    def _():
        write_partial(2 * wid + 1, seg_last)

    nout = jnp.where(fcount == 0, jnp.int32(0),
                     ((fcount - 1) & (RING - 1)) + 1)
    for i in range(RING):
        @pl.when(i < nout)
        def _():
            drain_one()


def _merge_body(part_hbm, out_hbm, mp, macc, fstage, sem):
    cc = lax.axis_index("c")
    sub = lax.axis_index("s")
    wid = sub * NC + cc

    @pl.when(wid == 0)
    def _():
        pltpu.sync_copy(part_hbm, mp)
        zero16 = jnp.zeros((16,), jnp.float32)
        for k in range(NK + 1):
            macc[pl.ds(16 * k, 16)] = zero16

        def finalize(seg):
            inv = 1.0 / macc[pl.ds(16 * NK, 16)]
            for k in range(NK):
                fstage[pl.ds(16 * k, 16)] = macc[pl.ds(16 * k, 16)] * inv
            pltpu.sync_copy(fstage, out_hbm.at[seg])
            for k in range(NK + 1):
                macc[pl.ds(16 * k, 16)] = zero16

        def rec_body(i, cur):
            segf = mp[i, pl.ds(16 * (NK + 1), 16)]
            seg = segf.astype(jnp.int32)[0]
            valid = seg >= 0
            newseg = valid & (seg != cur)

            @pl.when(newseg & (cur >= 0))
            def _():
                finalize(cur)

            @pl.when(valid)
            def _():
                for k in range(NK + 1):
                    sl = pl.ds(16 * k, 16)
                    macc[sl] = macc[sl] + mp[i, sl]

            return jnp.where(valid, seg, cur)

        cur = lax.fori_loop(0, 2 * NW, rec_body, jnp.int32(-1))

        @pl.when(cur >= 0)
        def _():
            finalize(cur)


@jax.jit
def _run(embedding, idxp, w_flat, b_vec):
    mesh = plsc.VectorSubcoreMesh(core_axis_name="c", subcore_axis_name="s")
    params = pltpu.CompilerParams(needs_layout_passes=False)
    main_k = pl.kernel(
        _main_body,
        out_type=jax.ShapeDtypeStruct((2 * NW, PW), jnp.float32),
        mesh=mesh,
        compiler_params=params,
        scratch_types=[
            pltpu.VMEM((2 * C, D), jnp.float32),
            pltpu.VMEM((2 * IB,), jnp.int32),
            pltpu.VMEM((D,), jnp.float32),
            pltpu.VMEM((16,), jnp.float32),
            pltpu.VMEM((RING, D), jnp.float32),
            pltpu.VMEM((D + 16,), jnp.float32),
            pltpu.VMEM((PW,), jnp.float32),
            pltpu.SemaphoreType.DMA,
            pltpu.SemaphoreType.DMA,
            pltpu.SemaphoreType.DMA,
        ],
    )
    merge_k = pl.kernel(
        _merge_body,
        out_type=(),
        mesh=mesh,
        compiler_params=params,
        scratch_types=[
            pltpu.VMEM((2 * NW, PW), jnp.float32),
            pltpu.VMEM((D + 16,), jnp.float32),
            pltpu.VMEM((D,), jnp.float32),
            pltpu.SemaphoreType.DMA,
        ],
    )
    out_ref = jax.new_ref(jnp.zeros((S, D), jnp.float32))
    part = main_k(embedding, idxp, w_flat, b_vec, out_ref)
    merge_k(part, out_ref)
    return out_ref[...]


def kernel(embedding, study_indexes, W, b):
    idx32 = study_indexes.astype(jnp.int32)
    idxp = jnp.concatenate([jnp.full((8,), -1, jnp.int32), idx32])
    w_flat = W.reshape(D)
    b_vec = jnp.broadcast_to(b.astype(jnp.float32), (16,))
    return _run(embedding, idxp, w_flat, b_vec)


# speedup vs baseline: 3.9422x; 1.0652x over previous
import functools

import jax
import jax.numpy as jnp
from jax import lax
from jax.experimental import pallas as pl
from jax.experimental.pallas import tpu as pltpu
from jax.experimental.pallas import tpu_sc as plsc

N = 160000
D = 256
S = 10000
NC = 2
NS = 16
NW = NC * NS
RPW = N // NW
C = 200
KCH = RPW // C
NG = C // 16
NK = D // 16
RING = 4
PW = D + 32
IB = C + 24


def _main_body(emb_hbm, idxp_hbm, w_hbm, b_hbm, out_hbm, part_hbm,
               ebuf, ibuf, tbuf, abuf, wbuf, bbuf, fbuf, accbuf, pstage,
               csem0, csem1, fsem):
    cc = lax.axis_index("c")
    sub = lax.axis_index("s")
    wid = sub * NC + cc
    row0 = wid * RPW

    pltpu.sync_copy(w_hbm, wbuf)
    pltpu.sync_copy(b_hbm, bbuf)

    bv = bbuf[...]
    wv = [wbuf[pl.ds(16 * k, 16)] for k in range(NK)]
    zero16 = jnp.zeros((16,), jnp.float32)
    pow2 = (jnp.int32(1) << jax.lax.broadcasted_iota(jnp.int32, (16,), 0))

    for k in range(NK + 1):
        accbuf[pl.ds(16 * k, 16)] = zero16

    def chunk_off(kc):
        return pl.multiple_of(row0 + kc * C, 8)

    def start_chunk(kc, par):
        p = chunk_off(kc)
        csemx = csem0 if par == 0 else csem1
        pltpu.async_copy(emb_hbm.at[pl.ds(p, C)],
                         ebuf.at[pl.ds(par * C, C)], csemx)
        pltpu.async_copy(idxp_hbm.at[pl.ds(p, C + 8)],
                         ibuf.at[pl.ds(par * IB, C + 8)], csemx)

    def wait_chunk(kc, par):
        p = chunk_off(kc)
        csemx = csem0 if par == 0 else csem1
        pltpu.make_async_copy(
            emb_hbm.at[pl.ds(p, C)], ebuf.at[pl.ds(par * C, C)], csemx).wait()
        pltpu.make_async_copy(
            idxp_hbm.at[pl.ds(p, C + 8)],
            ibuf.at[pl.ds(par * IB, C + 8)], csemx).wait()

    def drain_one():
        pltpu.make_async_copy(out_hbm.at[0], fbuf.at[0], fsem).wait()

    def reset_acc():
        for k in range(NK + 1):
            accbuf[pl.ds(16 * k, 16)] = zero16

    def write_partial(slot, seg):
        for k in range(NK + 1):
            pstage[pl.ds(16 * k, 16)] = accbuf[pl.ds(16 * k, 16)]
        pstage[pl.ds(16 * (NK + 1), 16)] = jnp.full(
            (16,), seg, jnp.int32).astype(jnp.float32)
        pltpu.sync_copy(pstage, part_hbm.at[slot])

    def flush_interior(fcount, seg):
        slot = fcount & (RING - 1)

        @pl.when((slot == 0) & (fcount > 0))
        def _():
            for _ in range(RING):
                drain_one()

        inv = 1.0 / accbuf[pl.ds(16 * NK, 16)]
        for k in range(NK):
            fbuf[slot, pl.ds(16 * k, 16)] = accbuf[pl.ds(16 * k, 16)] * inv
        pltpu.async_copy(fbuf.at[slot], out_hbm.at[seg], fsem)

    iota16 = jax.lax.broadcasted_iota(jnp.int32, (16,), 0)
    i16x16 = iota16 * 16

    def attn_pass(par, gi):
        base = 16 * gi
        row20 = par * C + base
        for j in range(16):
            row2 = row20 + j
            e = [ebuf[row2, pl.ds(16 * k, 16)] for k in range(NK)]
            d0 = e[0] * wv[0]
            d1 = e[1] * wv[1]
            d2 = e[2] * wv[2]
            d3 = e[3] * wv[3]
            for k in range(4, NK, 4):
                d0 = d0 + e[k] * wv[k]
                d1 = d1 + e[k + 1] * wv[k + 1]
                d2 = d2 + e[k + 2] * wv[k + 2]
                d3 = d3 + e[k + 3] * wv[k + 3]
            dvec = (d0 + d1) + (d2 + d3)
            plsc.store_scatter(tbuf, [i16x16 + j], dvec)
        tv = [tbuf[pl.ds(16 * k, 16)] for k in range(NK)]
        t0 = tv[0] + tv[1]
        t1 = tv[2] + tv[3]
        t2 = tv[4] + tv[5]
        t3 = tv[6] + tv[7]
        for k in range(8, NK, 8):
            t0 = t0 + (tv[k] + tv[k + 1])
            t1 = t1 + (tv[k + 2] + tv[k + 3])
            t2 = t2 + (tv[k + 4] + tv[k + 5])
            t3 = t3 + (tv[k + 6] + tv[k + 7])
        dots16 = (t0 + t1) + (t2 + t3)
        abuf[pl.ds(base, 16)] = jnp.exp(dots16 + bv)

    def accumulate_row(par, row):
        row2 = par * C + row
        attn = plsc.load_gather(abuf, [jnp.full((16,), row, jnp.int32)])
        e = [ebuf[row2, pl.ds(16 * k, 16)] for k in range(NK)]
        for k in range(NK):
            plsc.addupdate(accbuf.at[pl.ds(16 * k, 16)], attn * e[k])
        plsc.addupdate(accbuf.at[pl.ds(16 * NK, 16)], attn)

    UNROLL = 4

    def chunk_body(kc, st):
        nb, fcount = st
        par = kc & 1

        @pl.when(par == 0)
        def _():
            wait_chunk(kc, 0)

        @pl.when(par == 1)
        def _():
            wait_chunk(kc, 1)

        @pl.when(kc < KCH - 1)
        def _():
            @pl.when(par == 0)
            def _():
                start_chunk(kc + 1, 1)

            @pl.when(par == 1)
            def _():
                start_chunk(kc + 1, 0)

        def apass_body(gi, acc_):
            attn_pass(par, gi)
            return acc_

        lax.fori_loop(0, NG + 1, apass_body, jnp.int32(0))

        def group_body(gi, gst):
            base = 16 * gi
            av = ibuf[pl.ds(par * IB + base + 8, 16)]
            amv = ibuf[pl.ds(par * IB + base + 7, 16)]
            neq = av != amv
            bits_v = jnp.where(neq, pow2, jnp.int32(0))
            bits = jnp.sum(bits_v)
            bits = jnp.where(base + 16 <= C, bits, bits & 0xFF)

            def quad_body(rq, qst):
                nb, fcount = qst
                q0 = UNROLL * rq

                bjs, nbs, fcs = [], [], []
                nbq, fcq = nb, fcount
                for j in range(UNROLL):
                    bj = ((bits >> (q0 + j)) & 1) == 1
                    bjs.append(bj)
                    nbs.append(nbq)
                    fcs.append(fcq)
                    fcq = fcq + jnp.where(
                        bj & (nbq > 0), 1, 0).astype(jnp.int32)
                    nbq = nbq + jnp.where(bj, 1, 0).astype(jnp.int32)

                @pl.when(base + q0 < C)
                def _():
                    for j in range(UNROLL):
                        jj = q0 + j
                        row = base + jj

                        @pl.when(bjs[j])
                        def _():
                            seg = ibuf[pl.ds(par * IB + base + 7 + jj, 16)][0]

                            @pl.when(nbs[j] == 0)
                            def _():
                                write_partial(2 * wid, seg)

                            @pl.when(nbs[j] > 0)
                            def _():
                                flush_interior(fcs[j], seg)

                            reset_acc()

                        accumulate_row(par, row)

                nbq = jnp.where(base + q0 < C, nbq, nb)
                fcq = jnp.where(base + q0 < C, fcq, fcount)
                return nbq, fcq

            return lax.fori_loop(0, 16 // UNROLL, quad_body, gst)

        return lax.fori_loop(0, NG + 1, group_body, (nb, fcount))

    start_chunk(0, 0)
    nb, fcount = lax.fori_loop(0, KCH, chunk_body,
                               (jnp.int32(0), jnp.int32(0)))

    par_last = (KCH - 1) & 1
    seg_last = ibuf[pl.ds(par_last * IB + C - 8, 16)][15]

    @pl.when(nb == 0)
    def _():
        write_partial(2 * wid, seg_last)
        write_partial(2 * wid + 1, jnp.int32(-1))

    @pl.when(nb > 0)
    def _():
        write_partial(2 * wid + 1, seg_last)

    nout = jnp.where(fcount == 0, jnp.int32(0),
                     ((fcount - 1) & (RING - 1)) + 1)
    for i in range(RING):
        @pl.when(i < nout)
        def _():
            drain_one()


def _merge_body(part_hbm, out_hbm, mp, macc, fstage, sem):
    cc = lax.axis_index("c")
    sub = lax.axis_index("s")
    wid = sub * NC + cc

    @pl.when(wid == 0)
    def _():
        pltpu.sync_copy(part_hbm, mp)
        zero16 = jnp.zeros((16,), jnp.float32)
        for k in range(NK + 1):
            macc[pl.ds(16 * k, 16)] = zero16

        def finalize(seg):
            inv = 1.0 / macc[pl.ds(16 * NK, 16)]
            for k in range(NK):
                fstage[pl.ds(16 * k, 16)] = macc[pl.ds(16 * k, 16)] * inv
            pltpu.sync_copy(fstage, out_hbm.at[seg])
            for k in range(NK + 1):
                macc[pl.ds(16 * k, 16)] = zero16

        def rec_body(i, cur):
            segf = mp[i, pl.ds(16 * (NK + 1), 16)]
            seg = segf.astype(jnp.int32)[0]
            valid = seg >= 0
            newseg = valid & (seg != cur)

            @pl.when(newseg & (cur >= 0))
            def _():
                finalize(cur)

            @pl.when(valid)
            def _():
                for k in range(NK + 1):
                    sl = pl.ds(16 * k, 16)
                    macc[sl] = macc[sl] + mp[i, sl]

            return jnp.where(valid, seg, cur)

        cur = lax.fori_loop(0, 2 * NW, rec_body, jnp.int32(-1))

        @pl.when(cur >= 0)
        def _():
            finalize(cur)


@jax.jit
def _run(embedding, idxp, w_flat, b_vec):
    mesh = plsc.VectorSubcoreMesh(core_axis_name="c", subcore_axis_name="s")
    params = pltpu.CompilerParams(needs_layout_passes=False)
    main_k = pl.kernel(
        _main_body,
        out_type=jax.ShapeDtypeStruct((2 * NW, PW), jnp.float32),
        mesh=mesh,
        compiler_params=params,
        scratch_types=[
            pltpu.VMEM((2 * C + 16, D), jnp.float32),
            pltpu.VMEM((2 * IB,), jnp.int32),
            pltpu.VMEM((D,), jnp.float32),
            pltpu.VMEM((IB,), jnp.float32),
            pltpu.VMEM((D,), jnp.float32),
            pltpu.VMEM((16,), jnp.float32),
            pltpu.VMEM((RING, D), jnp.float32),
            pltpu.VMEM((D + 16,), jnp.float32),
            pltpu.VMEM((PW,), jnp.float32),
            pltpu.SemaphoreType.DMA,
            pltpu.SemaphoreType.DMA,
            pltpu.SemaphoreType.DMA,
        ],
    )
    merge_k = pl.kernel(
        _merge_body,
        out_type=(),
        mesh=mesh,
        compiler_params=params,
        scratch_types=[
            pltpu.VMEM((2 * NW, PW), jnp.float32),
            pltpu.VMEM((D + 16,), jnp.float32),
            pltpu.VMEM((D,), jnp.float32),
            pltpu.SemaphoreType.DMA,
        ],
    )
    out_ref = jax.new_ref(jnp.zeros((S, D), jnp.float32))
    part = main_k(embedding, idxp, w_flat, b_vec, out_ref)
    merge_k(part, out_ref)
    return out_ref[...]


def kernel(embedding, study_indexes, W, b):
    idx32 = study_indexes.astype(jnp.int32)
    idxp = jnp.concatenate([jnp.full((8,), -1, jnp.int32), idx32])
    w_flat = W.reshape(D)
    b_vec = jnp.broadcast_to(b.astype(jnp.float32), (16,))
    return _run(embedding, idxp, w_flat, b_vec)
